# baseline (speedup 1.0000x reference)
import jax
import jax.numpy as jnp
from jax import lax
from jax.experimental import pallas as pl
from jax.experimental.pallas import tpu as pltpu
from jax.experimental.pallas import tpu_sc as plsc

B = 16384
USER_DIM = 64
GENRE_DIM = 32
UVOC = 1001
GVOC = 21
GPI = 10
NC, NS, L = 2, 16, 16
NW = NC * NS
BPW = B // NW
NG = BPW // L


def _body(uid_hbm, gidT_hbm, utab_hbm, gtab_hbm, out_hbm,
          uidx_v, gidxT_v, utab_v, gtab_v, urows_v, gout_v):
  cid = lax.axis_index("c")
  sid = lax.axis_index("s")
  wid = sid * NC + cid
  base = wid * BPW

  pltpu.sync_copy(uid_hbm.at[wid], uidx_v)
  pltpu.sync_copy(gidT_hbm.at[wid], gidxT_v)
  pltpu.sync_copy(utab_hbm, utab_v)
  pltpu.sync_copy(gtab_hbm, gtab_v)

  scale = jnp.float32(1.0 / GPI)
  one = jnp.int32(1)
  iota = lax.iota(jnp.int32, L)

  def _grp(t, carry):
    item_rows = iota + t * L
    urow = uidx_v[pl.ds(t * L, L)] + one
    for c in range(USER_DIM):
      cvec = jnp.full((L,), c, jnp.int32)
      val = plsc.load_gather(utab_v, [urow, cvec])
      plsc.store_scatter(urows_v, [item_rows, cvec], val)
    rowsel = [gidxT_v[j, pl.ds(t * L, L)] for j in range(GPI)]
    for c in range(GENRE_DIM):
      cvec = jnp.full((L,), c, jnp.int32)
      acc = plsc.load_gather(gtab_v, [rowsel[0], cvec])
      for j in range(1, GPI):
        acc = acc + plsc.load_gather(gtab_v, [rowsel[j], cvec])
      plsc.store_scatter(gout_v, [item_rows, cvec], acc * scale)
    return carry

  lax.fori_loop(0, NG, _grp, 0)

  pltpu.sync_copy(urows_v, out_hbm.at[pl.ds(base, BPW), pl.ds(0, USER_DIM)])
  pltpu.sync_copy(gout_v,
                  out_hbm.at[pl.ds(base, BPW), pl.ds(USER_DIM, GENRE_DIM)])


@jax.jit
def kernel(user_id, movie_genres, user_table, genre_table):
  uid2 = user_id.reshape(NW, BPW)
  gidT = jnp.transpose(movie_genres.reshape(NW, BPW, GPI), (0, 2, 1))

  run = pl.kernel(
      _body,
      out_type=jax.ShapeDtypeStruct((B, USER_DIM + GENRE_DIM), jnp.float32),
      mesh=plsc.VectorSubcoreMesh(core_axis_name="c", subcore_axis_name="s",
                                  num_cores=NC, num_subcores=NS),
      scratch_types=[
          pltpu.VMEM((BPW,), jnp.int32),
          pltpu.VMEM((GPI, BPW), jnp.int32),
          pltpu.VMEM((UVOC, USER_DIM), jnp.float32),
          pltpu.VMEM((GVOC, GENRE_DIM), jnp.float32),
          pltpu.VMEM((BPW, USER_DIM), jnp.float32),
          pltpu.VMEM((BPW, GENRE_DIM), jnp.float32),
      ],
      compiler_params=pltpu.CompilerParams(use_tc_tiling_on_sc=False,
                                           needs_layout_passes=False),
  )
  return run(uid2, gidT, user_table, genre_table)


# speedup vs baseline: 1.2885x; 1.2885x over previous
import jax
import jax.numpy as jnp
from jax import lax
from jax.experimental import pallas as pl
from jax.experimental.pallas import tpu as pltpu
from jax.experimental.pallas import tpu_sc as plsc

B = 16384
USER_DIM = 64
GENRE_DIM = 32
UVOC = 1001
GVOC = 21
GPI = 10
NC, NS, L = 2, 16, 16
NW = NC * NS
BPW = B // NW
CH = 128
NCH = BPW // CH
NG = BPW // L


def _body(uid_hbm, gidT_hbm, utab_hbm, gtab_hbm, out_hbm,
          uidx_v, gidxT_v, gtab_v, urows_v, gout_v, utab_sh, usem):
  cid = lax.axis_index("c")
  sid = lax.axis_index("s")
  wid = sid * NC + cid
  base = wid * BPW

  @pl.when(sid == 0)
  def _():
    pltpu.sync_copy(utab_hbm, utab_sh)

  pltpu.sync_copy(uid_hbm.at[wid], uidx_v)
  pltpu.sync_copy(gidT_hbm.at[wid], gidxT_v)
  pltpu.sync_copy(gtab_hbm, gtab_v)

  for c in range(NCH):
    for t in range(CH // L):
      uidx_v[c, pl.ds(t * L, L)] = uidx_v[c, pl.ds(t * L, L)] + 1

  plsc.subcore_barrier()

  ucopies = [
      pltpu.async_copy(utab_sh.at[uidx_v.at[c]],
                       urows_v.at[pl.ds(c * CH, CH)], usem)
      for c in range(NCH)
  ]

  scale = jnp.float32(1.0 / GPI)
  iota = lax.iota(jnp.int32, L)

  def _grp(t, carry):
    rowsel = [gidxT_v[j, pl.ds(t * L, L)] for j in range(GPI)]
    item_rows = iota + t * L
    for c in range(GENRE_DIM):
      cvec = jnp.full((L,), c, jnp.int32)
      acc = plsc.load_gather(gtab_v, [rowsel[0], cvec])
      for j in range(1, GPI):
        acc = acc + plsc.load_gather(gtab_v, [rowsel[j], cvec])
      plsc.store_scatter(gout_v, [item_rows, cvec], acc * scale)
    return carry

  lax.fori_loop(0, NG, _grp, 0)

  for cp in ucopies:
    cp.wait()

  pltpu.sync_copy(urows_v, out_hbm.at[pl.ds(base, BPW), pl.ds(0, USER_DIM)])
  pltpu.sync_copy(gout_v,
                  out_hbm.at[pl.ds(base, BPW), pl.ds(USER_DIM, GENRE_DIM)])


@jax.jit
def kernel(user_id, movie_genres, user_table, genre_table):
  uid3 = user_id.reshape(NW, NCH, CH)
  gidT = jnp.transpose(movie_genres.reshape(NW, BPW, GPI), (0, 2, 1))

  run = pl.kernel(
      _body,
      out_type=jax.ShapeDtypeStruct((B, USER_DIM + GENRE_DIM), jnp.float32),
      mesh=plsc.VectorSubcoreMesh(core_axis_name="c", subcore_axis_name="s",
                                  num_cores=NC, num_subcores=NS),
      scratch_types=[
          pltpu.VMEM((NCH, CH), jnp.int32),
          pltpu.VMEM((GPI, BPW), jnp.int32),
          pltpu.VMEM((GVOC, GENRE_DIM), jnp.float32),
          pltpu.VMEM((BPW, USER_DIM), jnp.float32),
          pltpu.VMEM((BPW, GENRE_DIM), jnp.float32),
          pltpu.VMEM_SHARED((UVOC, USER_DIM), jnp.float32),
          pltpu.SemaphoreType.DMA,
      ],
      compiler_params=pltpu.CompilerParams(use_tc_tiling_on_sc=False,
                                           needs_layout_passes=False),
  )
  return run(uid3, gidT, user_table, genre_table)


# speedup vs baseline: 4.1404x; 3.2133x over previous
import functools

import jax
import jax.numpy as jnp
from jax import lax
from jax.experimental import pallas as pl
from jax.experimental.pallas import tpu as pltpu
from jax.experimental.pallas import tpu_sc as plsc

B = 16384
USER_DIM = 64
GENRE_DIM = 32
UVOC = 1001
GVOC = 21
GPI = 10
NC, NS, L = 2, 16, 16
NW = NC * NS
BPW = B // NW
CH = 128
NCH = BPW // CH
NG = BPW // L


def _body(uid_hbm, gidT_hbm, utab_hbm, gtab_hbm, out_hbm,
          uidx_v, gidxT_v, gtab_v, urows_v, gout_v, utab_sh, usem):
  cid = lax.axis_index("c")
  sid = lax.axis_index("s")
  wid = sid * NC + cid
  base = wid * BPW

  @pl.when(sid == 0)
  def _():
    pltpu.sync_copy(utab_hbm, utab_sh)

  pltpu.sync_copy(uid_hbm.at[wid], uidx_v)
  pltpu.sync_copy(gidT_hbm.at[wid], gidxT_v)
  pltpu.sync_copy(gtab_hbm, gtab_v)

  for c in range(NCH):
    for t in range(CH // L):
      uidx_v[c, pl.ds(t * L, L)] = uidx_v[c, pl.ds(t * L, L)] + 1

  plsc.subcore_barrier()

  ucopies = [
      pltpu.async_copy(utab_sh.at[uidx_v.at[c]],
                       urows_v.at[pl.ds(c * CH, CH)], usem)
      for c in range(NCH)
  ]

  scale = jnp.float32(1.0 / GPI)
  iota = lax.iota(jnp.int32, L)

  cvecs = [jnp.full((L,), c, jnp.int32) for c in range(GENRE_DIM)]

  @functools.partial(plsc.parallel_loop, 0, NG, unroll=2)
  def _grp(t):
    rowsel = [gidxT_v[j, pl.ds(t * L, L)] for j in range(GPI)]
    item_rows = iota + t * L
    accs = [plsc.load_gather(gtab_v, [rowsel[0], cvecs[c]])
            for c in range(GENRE_DIM)]
    for j in range(1, GPI):
      accs = [a + plsc.load_gather(gtab_v, [rowsel[j], cvecs[c]])
              for c, a in enumerate(accs)]
    for c in range(GENRE_DIM):
      plsc.store_scatter(gout_v, [item_rows, cvecs[c]], accs[c] * scale)

  for cp in ucopies:
    cp.wait()

  pltpu.sync_copy(urows_v, out_hbm.at[pl.ds(base, BPW), pl.ds(0, USER_DIM)])
  pltpu.sync_copy(gout_v,
                  out_hbm.at[pl.ds(base, BPW), pl.ds(USER_DIM, GENRE_DIM)])


@jax.jit
def kernel(user_id, movie_genres, user_table, genre_table):
  uid3 = user_id.reshape(NW, NCH, CH)
  gidT = jnp.transpose(movie_genres.reshape(NW, BPW, GPI), (0, 2, 1))

  run = pl.kernel(
      _body,
      out_type=jax.ShapeDtypeStruct((B, USER_DIM + GENRE_DIM), jnp.float32),
      mesh=plsc.VectorSubcoreMesh(core_axis_name="c", subcore_axis_name="s",
                                  num_cores=NC, num_subcores=NS),
      scratch_types=[
          pltpu.VMEM((NCH, CH), jnp.int32),
          pltpu.VMEM((GPI, BPW), jnp.int32),
          pltpu.VMEM((GVOC, GENRE_DIM), jnp.float32),
          pltpu.VMEM((BPW, USER_DIM), jnp.float32),
          pltpu.VMEM((BPW, GENRE_DIM), jnp.float32),
          pltpu.VMEM_SHARED((UVOC, USER_DIM), jnp.float32),
          pltpu.SemaphoreType.DMA,
      ],
      compiler_params=pltpu.CompilerParams(use_tc_tiling_on_sc=False,
                                           needs_layout_passes=False),
  )
  return run(uid3, gidT, user_table, genre_table)
